# baseline (speedup 1.0000x reference)
import functools

import jax
import jax.numpy as jnp
from jax import lax
from jax.experimental import pallas as pl
from jax.experimental.pallas import tpu as pltpu
from jax.experimental.pallas import tpu_sc as plsc

N_NODES = 10000
D = 128

NC = 2
NS = 16
NW = NC * NS

C_OP = 128
N_ACC = 10240
ROWS_PER_TILE = N_ACC // NS


def _fill_f32(ref, nrows, ncols, value):
    def body(r, c):
        for j in range(ncols // 16):
            ref[r, pl.ds(j * 16, 16)] = jnp.full((16,), value, jnp.float32)
        return c
    lax.fori_loop(0, nrows, body, 0)


def _make_deg_kernel(n_ops):
    mesh = plsc.VectorSubcoreMesh(core_axis_name="c", subcore_axis_name="s")

    @functools.partial(
        pl.kernel,
        out_type=jax.ShapeDtypeStruct((NC * N_ACC, D), jnp.float32),
        mesh=mesh,
        scratch_types=[
            pltpu.VMEM((n_ops, C_OP), jnp.int32),
            pltpu.VMEM((C_OP, D), jnp.float32),
            pltpu.VMEM_SHARED((N_ACC, D), jnp.float32),
        ],
    )
    def deg_kernel(dst_hbm, out_hbm, dstv, ones, acc):
        cid = lax.axis_index("c")
        sid = lax.axis_index("s")
        wid = cid * NS + sid
        _fill_f32(ones, C_OP, D, 0.0)
        for t in range(ROWS_PER_TILE // C_OP):
            pltpu.sync_copy(ones, acc.at[pl.ds(sid * ROWS_PER_TILE + t * C_OP, C_OP)])
        plsc.subcore_barrier()
        _fill_f32(ones, C_OP, D, 1.0)
        pltpu.sync_copy(dst_hbm.at[wid], dstv)

        def step(j, c):
            pltpu.sync_copy(ones, acc.at[dstv.at[j]], add=True)
            return c
        lax.fori_loop(0, n_ops, step, 0)
        plsc.subcore_barrier()
        for t in range(ROWS_PER_TILE // C_OP):
            r0 = sid * ROWS_PER_TILE + t * C_OP
            pltpu.sync_copy(acc.at[pl.ds(r0, C_OP)], ones)
            pltpu.sync_copy(ones, out_hbm.at[pl.ds(cid * N_ACC + r0, C_OP)])

    return deg_kernel


IB = 8
NBUF = 2


def _make_agg_kernel(n0, n1):
    assert n0 % IB == 0 and n1 % IB == 0 and IB % NBUF == 0
    mesh = plsc.VectorSubcoreMesh(core_axis_name="c", subcore_axis_name="s")

    scratch = (
        [pltpu.VMEM((IB, C_OP), jnp.int32)] * 2
        + [pltpu.VMEM((C_OP, D), jnp.float32)] * NBUF
        + [pltpu.VMEM_SHARED((N_ACC, D), jnp.float32)]
        + [pltpu.SemaphoreType.DMA] * NBUF
    )

    @functools.partial(
        pl.kernel,
        out_type=jax.ShapeDtypeStruct((NC * N_ACC, D), jnp.float32),
        mesh=mesh,
        scratch_types=scratch,
    )
    def agg_kernel(hs_hbm, src_hbm, dst_hbm, out_hbm, srcv, dstv, *bufs):
        rows = bufs[:NBUF]
        acc = bufs[NBUF]
        sems = bufs[NBUF + 1:]
        cid = lax.axis_index("c")
        sid = lax.axis_index("s")
        _fill_f32(rows[0], C_OP, D, 0.0)
        for t in range(ROWS_PER_TILE // C_OP):
            pltpu.sync_copy(rows[0], acc.at[pl.ds(sid * ROWS_PER_TILE + t * C_OP, C_OP)])
        plsc.subcore_barrier()
        nch0, nch1 = n0 // IB, n1 // IB
        my_nch = jnp.where(cid == 0, nch0, nch1)
        base = jnp.where(cid == 0, sid * nch0, NS * nch0 + sid * nch1)

        def chunk(ci, c):
            pltpu.sync_copy(src_hbm.at[base + ci], srcv)
            pltpu.sync_copy(dst_hbm.at[base + ci], dstv)
            for b in range(NBUF):
                pltpu.async_copy(hs_hbm.at[srcv.at[b]], rows[b], sems[b])

            def round_body(r, c2):
                for b in range(NBUF):
                    j = r * NBUF + b
                    pltpu.make_async_copy(hs_hbm.at[srcv.at[j]], rows[b], sems[b]).wait()
                    pltpu.sync_copy(rows[b], acc.at[dstv.at[j]], add=True)
                    jn = lax.min(j + NBUF, IB - 1)
                    pltpu.async_copy(hs_hbm.at[srcv.at[jn]], rows[b], sems[b])
                return c2
            lax.fori_loop(0, IB // NBUF, round_body, 0)
            for b in range(NBUF):
                pltpu.make_async_copy(hs_hbm.at[srcv.at[0]], rows[b], sems[b]).wait()
            return c
        lax.fori_loop(0, my_nch, chunk, 0)
        plsc.subcore_barrier()
        for t in range(ROWS_PER_TILE // C_OP):
            r0 = sid * ROWS_PER_TILE + t * C_OP
            pltpu.sync_copy(acc.at[pl.ds(r0, C_OP)], rows[0])
            pltpu.sync_copy(rows[0], out_hbm.at[pl.ds(cid * N_ACC + r0, C_OP)])

    return agg_kernel


def _dis_body(cnt_ref, out_ref):
    d = cnt_ref[0:N_ACC, :] + cnt_ref[N_ACC:2 * N_ACC, :] + 1.0
    out_ref[...] = lax.rsqrt(d)


def _dis_tc(counts):
    return pl.pallas_call(
        _dis_body,
        out_shape=jax.ShapeDtypeStruct((N_ACC, D), jnp.float32),
    )(counts)


_BLK = 1000
_GRID = N_NODES // _BLK


def _mm_scale_body(x_ref, w_ref, dis_ref, out_ref):
    h = jnp.dot(x_ref[...], w_ref[...], preferred_element_type=jnp.float32)
    out_ref[...] = h * dis_ref[...]


def _tc1(x, W, dis):
    return pl.pallas_call(
        _mm_scale_body,
        grid=(_GRID,),
        in_specs=[
            pl.BlockSpec((_BLK, D), lambda i: (i, 0)),
            pl.BlockSpec((D, D), lambda i: (0, 0)),
            pl.BlockSpec((_BLK, 1), lambda i: (i, 0)),
        ],
        out_specs=pl.BlockSpec((_BLK, D), lambda i: (i, 0)),
        out_shape=jax.ShapeDtypeStruct((N_NODES, D), jnp.float32),
    )(x, W, dis)


def _mid_body(a0_ref, a1_ref, hs_ref, dis_ref, b_ref, w_ref, out_ref):
    dis = dis_ref[...]
    v = (a0_ref[...] + a1_ref[...] + hs_ref[...]) * dis + b_ref[...]
    t = jnp.maximum(v, 0.0) * dis
    out_ref[...] = jnp.dot(t, w_ref[...], preferred_element_type=jnp.float32)


def _tc2(a0, a1, hs, dis, b, W):
    return pl.pallas_call(
        _mid_body,
        grid=(_GRID,),
        in_specs=[
            pl.BlockSpec((_BLK, D), lambda i: (i, 0)),
            pl.BlockSpec((_BLK, D), lambda i: (i, 0)),
            pl.BlockSpec((_BLK, D), lambda i: (i, 0)),
            pl.BlockSpec((_BLK, 1), lambda i: (i, 0)),
            pl.BlockSpec((1, D), lambda i: (0, 0)),
            pl.BlockSpec((D, D), lambda i: (0, 0)),
        ],
        out_specs=pl.BlockSpec((_BLK, D), lambda i: (i, 0)),
        out_shape=jax.ShapeDtypeStruct((N_NODES, D), jnp.float32),
    )(a0, a1, hs, dis, b, W)


def _final_body(a0_ref, a1_ref, hs_ref, dis_ref, b_ref, out_ref):
    v = (a0_ref[...] + a1_ref[...] + hs_ref[...]) * dis_ref[...] + b_ref[...]
    m = jnp.max(v, axis=1, keepdims=True)
    z = v - m
    out_ref[...] = z - jnp.log(jnp.sum(jnp.exp(z), axis=1, keepdims=True))


def _tc3(a0, a1, hs, dis, b):
    return pl.pallas_call(
        _final_body,
        grid=(_GRID,),
        in_specs=[
            pl.BlockSpec((_BLK, D), lambda i: (i, 0)),
            pl.BlockSpec((_BLK, D), lambda i: (i, 0)),
            pl.BlockSpec((_BLK, D), lambda i: (i, 0)),
            pl.BlockSpec((_BLK, 1), lambda i: (i, 0)),
            pl.BlockSpec((1, D), lambda i: (0, 0)),
        ],
        out_specs=pl.BlockSpec((_BLK, D), lambda i: (i, 0)),
        out_shape=jax.ShapeDtypeStruct((N_NODES, D), jnp.float32),
    )(a0, a1, hs, dis, b)


_F0 = 0.75


def kernel(x, edge_index, W1, b1, W2, b2):
    src = edge_index[0].astype(jnp.int32)
    dst = edge_index[1].astype(jnp.int32)
    E = src.shape[0]
    n_ops = -(-E // (NW * C_OP))
    n_ops = -(-n_ops // (IB // NC)) * (IB // NC)
    n_pair = NC * n_ops
    n0 = min(n_pair, max(0, round(n_pair * _F0 / IB) * IB))
    n1 = n_pair - n0
    e_pad = NW * C_OP * n_ops
    pad = e_pad - E
    src_f = jnp.concatenate([src, jnp.zeros((pad,), jnp.int32)]).reshape(-1, C_OP)
    dst_f = jnp.concatenate([dst, jnp.full((pad,), N_ACC - 1, jnp.int32)]).reshape(-1, C_OP)
    src_p = src_f.reshape(-1, IB, C_OP)
    dst_p = dst_f.reshape(-1, IB, C_OP)
    dst_d = dst_f.reshape(NW, n_ops, C_OP)

    counts = _make_deg_kernel(n_ops)(dst_d)
    dis_full = _dis_tc(counts)
    dis = dis_full[:N_NODES, 0:1]

    agg = _make_agg_kernel(n0, n1)

    hs1 = _tc1(x, W1, dis)
    p1 = agg(hs1, src_p, dst_p)
    hs2 = _tc2(p1[:N_NODES], p1[N_ACC:N_ACC + N_NODES], hs1, dis,
               b1.reshape(1, D), W2)
    p2 = agg(hs2, src_p, dst_p)
    return _tc3(p2[:N_NODES], p2[N_ACC:N_ACC + N_NODES], hs2, dis,
                b2.reshape(1, D))


# speedup vs baseline: 1.7415x; 1.7415x over previous
import functools

import jax
import jax.numpy as jnp
from jax import lax
from jax.experimental import pallas as pl
from jax.experimental.pallas import tpu as pltpu
from jax.experimental.pallas import tpu_sc as plsc

N_NODES = 10000
D = 128

NC = 2
NS = 16
NW = NC * NS

C_OP = 128
N_ACC = 10240
ROWS_PER_TILE = N_ACC // NS


def _fill_f32(ref, nrows, ncols, value):
    def body(r, c):
        for j in range(ncols // 16):
            ref[r, pl.ds(j * 16, 16)] = jnp.full((16,), value, jnp.float32)
        return c
    lax.fori_loop(0, nrows, body, 0)


def _make_deg_kernel(n_ops):
    mesh = plsc.VectorSubcoreMesh(core_axis_name="c", subcore_axis_name="s")

    @functools.partial(
        pl.kernel,
        out_type=jax.ShapeDtypeStruct((NC * N_ACC, D), jnp.float32),
        mesh=mesh,
        scratch_types=[
            pltpu.VMEM((n_ops, C_OP), jnp.int32),
            pltpu.VMEM((C_OP, D), jnp.float32),
            pltpu.VMEM_SHARED((N_ACC, D), jnp.float32),
            pltpu.SemaphoreType.DMA,
        ],
    )
    def deg_kernel(dst_hbm, out_hbm, dstv, ones, acc, sem):
        cid = lax.axis_index("c")
        sid = lax.axis_index("s")
        wid = cid * NS + sid
        _fill_f32(ones, C_OP, D, 0.0)
        for t in range(ROWS_PER_TILE // C_OP):
            pltpu.sync_copy(ones, acc.at[pl.ds(sid * ROWS_PER_TILE + t * C_OP, C_OP)])
        plsc.subcore_barrier()
        _fill_f32(ones, C_OP, D, 1.0)
        pltpu.sync_copy(dst_hbm.at[wid], dstv)

        k = 8
        def group(g, c):
            for i in range(k):
                pltpu.async_copy(ones, acc.at[dstv.at[g * k + i]], sem)
            for i in range(k):
                pltpu.make_async_copy(ones, acc.at[dstv.at[0]], sem).wait()
            return c
        lax.fori_loop(0, n_ops // k, group, 0)

        def step(j, c):
            pltpu.sync_copy(ones, acc.at[dstv.at[j]], add=True)
            return c
        lax.fori_loop(n_ops // k * k, n_ops, step, 0)
        plsc.subcore_barrier()
        for t in range(ROWS_PER_TILE // C_OP):
            r0 = sid * ROWS_PER_TILE + t * C_OP
            pltpu.sync_copy(acc.at[pl.ds(r0, C_OP)], ones)
            pltpu.sync_copy(ones, out_hbm.at[pl.ds(cid * N_ACC + r0, C_OP)])

    return deg_kernel


def _make_agg_kernel(n0, n1):
    max_ops = max(n0, n1)
    mesh = plsc.VectorSubcoreMesh(core_axis_name="c", subcore_axis_name="s")

    @functools.partial(
        pl.kernel,
        out_type=jax.ShapeDtypeStruct((NC * N_ACC, D), jnp.float32),
        mesh=mesh,
        scratch_types=[
            pltpu.VMEM((max_ops, C_OP), jnp.int32),
            pltpu.VMEM((max_ops, C_OP), jnp.int32),
            pltpu.VMEM((C_OP, D), jnp.float32),
            pltpu.VMEM_SHARED((N_ACC, D), jnp.float32),
            pltpu.SemaphoreType.DMA,
        ],
    )
    def agg_kernel(hs_hbm, src_hbm, dst_hbm, out_hbm, srcv, dstv, rows, acc, sem):
        cid = lax.axis_index("c")
        sid = lax.axis_index("s")
        wid = cid * NS + sid
        _fill_f32(rows, C_OP, D, 0.0)
        for t in range(ROWS_PER_TILE // C_OP):
            pltpu.sync_copy(rows, acc.at[pl.ds(sid * ROWS_PER_TILE + t * C_OP, C_OP)])
        plsc.subcore_barrier()
        pltpu.sync_copy(src_hbm.at[wid], srcv)
        pltpu.sync_copy(dst_hbm.at[wid], dstv)
        my_ops = jnp.where(cid == 0, n0, n1)

        def step(j, c):
            pltpu.async_copy(hs_hbm.at[srcv.at[j]], rows, sem).wait()
            pltpu.sync_copy(rows, acc.at[dstv.at[j]], add=True)
            return c
        lax.fori_loop(0, my_ops, step, 0)
        plsc.subcore_barrier()
        for t in range(ROWS_PER_TILE // C_OP):
            r0 = sid * ROWS_PER_TILE + t * C_OP
            pltpu.sync_copy(acc.at[pl.ds(r0, C_OP)], rows)
            pltpu.sync_copy(rows, out_hbm.at[pl.ds(cid * N_ACC + r0, C_OP)])

    return agg_kernel


def _dis_body(cnt_ref, out_ref):
    d = cnt_ref[0:N_ACC, :] + cnt_ref[N_ACC:2 * N_ACC, :] + 1.0
    out_ref[...] = lax.rsqrt(d)


def _dis_tc(counts):
    return pl.pallas_call(
        _dis_body,
        out_shape=jax.ShapeDtypeStruct((N_ACC, D), jnp.float32),
    )(counts)


_BLK = 1000
_GRID = N_NODES // _BLK


def _mm_scale_body(x_ref, w_ref, dis_ref, out_ref):
    h = jnp.dot(x_ref[...], w_ref[...], preferred_element_type=jnp.float32)
    out_ref[...] = h * dis_ref[...]


def _tc1(x, W, dis):
    return pl.pallas_call(
        _mm_scale_body,
        grid=(_GRID,),
        in_specs=[
            pl.BlockSpec((_BLK, D), lambda i: (i, 0)),
            pl.BlockSpec((D, D), lambda i: (0, 0)),
            pl.BlockSpec((_BLK, 1), lambda i: (i, 0)),
        ],
        out_specs=pl.BlockSpec((_BLK, D), lambda i: (i, 0)),
        out_shape=jax.ShapeDtypeStruct((N_NODES, D), jnp.float32),
    )(x, W, dis)


def _mid_body(a0_ref, a1_ref, hs_ref, dis_ref, b_ref, w_ref, out_ref):
    dis = dis_ref[...]
    v = (a0_ref[...] + a1_ref[...] + hs_ref[...]) * dis + b_ref[...]
    t = jnp.maximum(v, 0.0) * dis
    out_ref[...] = jnp.dot(t, w_ref[...], preferred_element_type=jnp.float32)


def _tc2(a0, a1, hs, dis, b, W):
    return pl.pallas_call(
        _mid_body,
        grid=(_GRID,),
        in_specs=[
            pl.BlockSpec((_BLK, D), lambda i: (i, 0)),
            pl.BlockSpec((_BLK, D), lambda i: (i, 0)),
            pl.BlockSpec((_BLK, D), lambda i: (i, 0)),
            pl.BlockSpec((_BLK, 1), lambda i: (i, 0)),
            pl.BlockSpec((1, D), lambda i: (0, 0)),
            pl.BlockSpec((D, D), lambda i: (0, 0)),
        ],
        out_specs=pl.BlockSpec((_BLK, D), lambda i: (i, 0)),
        out_shape=jax.ShapeDtypeStruct((N_NODES, D), jnp.float32),
    )(a0, a1, hs, dis, b, W)


def _final_body(a0_ref, a1_ref, hs_ref, dis_ref, b_ref, out_ref):
    v = (a0_ref[...] + a1_ref[...] + hs_ref[...]) * dis_ref[...] + b_ref[...]
    m = jnp.max(v, axis=1, keepdims=True)
    z = v - m
    out_ref[...] = z - jnp.log(jnp.sum(jnp.exp(z), axis=1, keepdims=True))


def _tc3(a0, a1, hs, dis, b):
    return pl.pallas_call(
        _final_body,
        grid=(_GRID,),
        in_specs=[
            pl.BlockSpec((_BLK, D), lambda i: (i, 0)),
            pl.BlockSpec((_BLK, D), lambda i: (i, 0)),
            pl.BlockSpec((_BLK, D), lambda i: (i, 0)),
            pl.BlockSpec((_BLK, 1), lambda i: (i, 0)),
            pl.BlockSpec((1, D), lambda i: (0, 0)),
        ],
        out_specs=pl.BlockSpec((_BLK, D), lambda i: (i, 0)),
        out_shape=jax.ShapeDtypeStruct((N_NODES, D), jnp.float32),
    )(a0, a1, hs, dis, b)


_F0 = 0.75


def _pack_tiles(arr, fillval, n0, n1):
    m = max(n0, n1)
    p0 = arr[:NS * n0].reshape(NS, n0, C_OP)
    p1 = arr[NS * n0:].reshape(NS, n1, C_OP)
    p0 = jnp.pad(p0, ((0, 0), (0, m - n0), (0, 0)), constant_values=fillval)
    p1 = jnp.pad(p1, ((0, 0), (0, m - n1), (0, 0)), constant_values=fillval)
    return jnp.concatenate([p0, p1])


def kernel(x, edge_index, W1, b1, W2, b2):
    src = edge_index[0].astype(jnp.int32)
    dst = edge_index[1].astype(jnp.int32)
    E = src.shape[0]
    n_ops = -(-E // (NW * C_OP))
    n_pair = NC * n_ops
    n0 = min(n_pair, max(0, round(n_pair * _F0)))
    n1 = n_pair - n0
    e_pad = NW * C_OP * n_ops
    pad = e_pad - E
    src_f = jnp.concatenate([src, jnp.zeros((pad,), jnp.int32)]).reshape(-1, C_OP)
    dst_f = jnp.concatenate([dst, jnp.full((pad,), N_ACC - 1, jnp.int32)]).reshape(-1, C_OP)
    src_p = _pack_tiles(src_f, 0, n0, n1)
    dst_p = _pack_tiles(dst_f, N_ACC - 1, n0, n1)
    dst_d = dst_f.reshape(NW, n_ops, C_OP)

    counts = _make_deg_kernel(n_ops)(dst_d)
    dis_full = _dis_tc(counts)
    dis = dis_full[:N_NODES, 0:1]

    agg = _make_agg_kernel(n0, n1)

    hs1 = _tc1(x, W1, dis)
    p1 = agg(hs1, src_p, dst_p)
    hs2 = _tc2(p1[:N_NODES], p1[N_ACC:N_ACC + N_NODES], hs1, dis,
               b1.reshape(1, D), W2)
    p2 = agg(hs2, src_p, dst_p)
    return _tc3(p2[:N_NODES], p2[N_ACC:N_ACC + N_NODES], hs2, dis,
                b2.reshape(1, D))
